# baseline (speedup 1.0000x reference)
import functools

import jax
import jax.numpy as jnp
from jax import lax
from jax.experimental import pallas as pl
from jax.experimental.pallas import tpu as pltpu
from jax.experimental.pallas import tpu_sc as plsc

EPS = 1e-5
_NW = 32
_FPAD = 32
_NG = 4


_SC_MESH = plsc.VectorSubcoreMesh(core_axis_name="c", subcore_axis_name="s")
_SC_PARAMS = pltpu.CompilerParams(
    use_tc_tiling_on_sc=False, needs_layout_passes=False
)


def _sc_indices(catT, F, vpad):
    B = catT.shape[1]
    per_b = B // 8
    nidx = B * _FPAD

    @functools.partial(
        pl.kernel,
        mesh=_SC_MESH,
        compiler_params=_SC_PARAMS,
        out_type=jax.ShapeDtypeStruct((nidx,), jnp.int32),
        scratch_types=[
            pltpu.VMEM((8, per_b), jnp.int32),
            pltpu.VMEM((8 * per_b,), jnp.int32),
        ],
    )
    def idx_k(cat_hbm, out_hbm, cat_v, idx_v):
        wid = lax.axis_index("s") * 2 + lax.axis_index("c")
        k = wid // 8
        b0 = (wid % 8) * per_b
        frs = []
        for fj in range(8):
            fr = 8 * k + fj
            fr = jnp.where(fr < F, fr, (F - 2) + (fj & 1))
            frs.append(fr)
            pltpu.sync_copy(cat_hbm.at[fr, pl.ds(b0, per_b)], cat_v.at[fj])
        lanes = lax.iota(jnp.int32, 16)

        def body(cch, carry):
            for fj in range(8):
                v = cat_v[fj, pl.ds(cch * 16, 16)]
                u = v & 1023
                fl = frs[fj] - (k // 2) * 16
                t = (v - u) + ((v & 127) << 3) + (u >> 7) + fl * vpad
                pos = lanes * 8 + (cch * 128 + fj)
                plsc.store_scatter(idx_v, [pos], t)
            return carry

        lax.fori_loop(0, per_b // 16, body, 0)
        pltpu.sync_copy(idx_v, out_hbm.at[pl.ds(wid * 8 * per_b, 8 * per_b)])

    return idx_k(catT)


def _sc_gather(table, idx, C, nidx, gbase):
    Dd = table.shape[1]
    per_w = nidx // _NW
    n_chunk = per_w // C

    @functools.partial(
        pl.kernel,
        mesh=_SC_MESH,
        compiler_params=_SC_PARAMS,
        out_type=jax.ShapeDtypeStruct((nidx, Dd), jnp.bfloat16),
        scratch_types=[
            pltpu.VMEM((per_w,), jnp.int32),
            pltpu.VMEM((2, C, Dd), jnp.bfloat16),
            pltpu.SemaphoreType.DMA,
            pltpu.SemaphoreType.DMA,
        ],
    )
    def gather_k(table_hbm, idx_hbm, out_hbm, idx_v, rows_v, gsem0, gsem1):
        wid = lax.axis_index("s") * 2 + lax.axis_index("c")
        base = wid * per_w
        pltpu.sync_copy(idx_hbm.at[pl.ds(gbase + base, per_w)], idx_v)
        sems = (gsem0, gsem1)
        cur = pltpu.async_copy(
            table_hbm.at[idx_v.at[pl.ds(0, C)]], rows_v.at[0], sems[0]
        )
        for j in range(n_chunk):
            nxt = None
            if j + 1 < n_chunk:
                nxt = pltpu.async_copy(
                    table_hbm.at[idx_v.at[pl.ds((j + 1) * C, C)]],
                    rows_v.at[(j + 1) % 2],
                    sems[(j + 1) % 2],
                )
            cur.wait()
            pltpu.sync_copy(rows_v.at[j % 2], out_hbm.at[pl.ds(base + j * C, C)])
            cur = nxt

    return gather_k(table, idx)


_VC = 50176


def _tr_body(tt_ref, out_ref):
    x = tt_ref[0]
    for c in range(_VC // 1024):
        w = jnp.concatenate(
            [x[:, c * 1024 + q * 128 : c * 1024 + (q + 1) * 128] for q in range(8)],
            axis=0,
        )
        out_ref[pl.ds(c * 128, 128), :] = w.T.astype(jnp.bfloat16)


def _num_stats_body(numT_ref, acc_ref):
    x = numT_ref[...]
    stats = jnp.concatenate(
        [jnp.sum(x, axis=1, keepdims=True), jnp.sum(x * x, axis=1, keepdims=True)],
        axis=1,
    )
    i = pl.program_id(0)

    @pl.when(i == 0)
    def _():
        acc_ref[...] = stats

    @pl.when(i > 0)
    def _():
        acc_ref[...] += stats


def _l1_body(e0_ref, e1_ref, e2_ref, e3_ref, numT_ref, nst_ref, g0_ref,
             be0_ref, w1_ref, w1n_ref, b1_ref, h1_ref, acc_ref, *, nB):
    i = pl.program_id(0)
    m = nst_ref[:, 0:1] * (1.0 / nB)
    var = nst_ref[:, 1:2] * (1.0 / nB) - m * m
    numn = (numT_ref[...] - m) * lax.rsqrt(var + EPS) * g0_ref[...] + be0_ref[...]
    h = lax.dot_general(
        numn, w1n_ref[...], (((0,), (0,)), ((), ())),
        preferred_element_type=jnp.float32,
    ) + b1_ref[...]
    for k, e_ref in enumerate((e0_ref, e1_ref, e2_ref, e3_ref)):
        h += jnp.dot(
            e_ref[...], w1_ref[pl.ds(k * 128, 128), :],
            preferred_element_type=jnp.float32,
        )
    h = jnp.maximum(h, 0.0)
    h1_ref[...] = h
    stats = jnp.concatenate(
        [jnp.sum(h, axis=0, keepdims=True), jnp.sum(h * h, axis=0, keepdims=True)],
        axis=0,
    )

    @pl.when(i == 0)
    def _():
        acc_ref[...] = stats

    @pl.when(i > 0)
    def _():
        acc_ref[...] += stats


def _l2_body(h1_ref, st_ref, g_ref, be_ref, w2_ref, b2_ref, h2_ref, acc_ref, *, nB):
    i = pl.program_id(0)
    mean = st_ref[0:1, :] * (1.0 / nB)
    var = st_ref[1:2, :] * (1.0 / nB) - mean * mean
    xn = (h1_ref[...] - mean) * lax.rsqrt(var + EPS) * g_ref[...] + be_ref[...]
    h = jnp.dot(xn, w2_ref[...], preferred_element_type=jnp.float32)
    h = jnp.maximum(h + b2_ref[...], 0.0)
    h2_ref[...] = h
    stats = jnp.concatenate(
        [jnp.sum(h, axis=0, keepdims=True), jnp.sum(h * h, axis=0, keepdims=True)],
        axis=0,
    )

    @pl.when(i == 0)
    def _():
        acc_ref[...] = stats

    @pl.when(i > 0)
    def _():
        acc_ref[...] += stats


def _l3_body(h2_ref, st_ref, g_ref, be_ref, wo_ref, bo_ref, out_ref, *, nB):
    mean = st_ref[0:1, :] * (1.0 / nB)
    var = st_ref[1:2, :] * (1.0 / nB) - mean * mean
    xn = (h2_ref[...] - mean) * lax.rsqrt(var + EPS) * g_ref[...] + be_ref[...]
    out_ref[...] = lax.dot_general(
        wo_ref[...], xn, (((1,), (1,)), ((), ())),
        preferred_element_type=jnp.float32,
    ) + bo_ref[...]


def kernel(numerical_data, cat_data, tables, W1, b1, W2, b2, Wo, bo,
           g0, be0, g1, be1, g2, be2):
    B, NUM = numerical_data.shape
    F = cat_data.shape[1]
    V = tables.shape[1]
    D = tables.shape[2]
    ED = F * D
    GW = _FPAD // _NG
    H1, H2 = W1.shape[0], W2.shape[0]
    fB = float(B)

    tt = jnp.swapaxes(tables, 1, 2)
    nch = (V + _VC - 1) // _VC
    vpad = nch * _VC

    catT = jnp.swapaxes(cat_data, 0, 1)
    idx_r = _sc_indices(catT, F, vpad)

    halves = []
    for h in range(2):
        f0 = h * 2 * GW
        nf = min(2 * GW, F - f0)
        tp = pl.pallas_call(
            _tr_body,
            grid=(nf, nch),
            in_specs=[
                pl.BlockSpec((1, D, _VC), lambda f, c, f0=f0: (f0 + f, 0, c))
            ],
            out_specs=pl.BlockSpec((_VC // 8, 128), lambda f, c: (f * nch + c, 0)),
            out_shape=jax.ShapeDtypeStruct((nf * vpad // 8, 128), jnp.bfloat16),
        )(tt)
        emb_h = _sc_gather(
            tp.reshape(nf * vpad, D), idx_r, C=2048,
            nidx=2 * B * GW, gbase=h * 2 * B * GW,
        )
        halves.append(emb_h.reshape(2 * B, GW * D))
    embs = [halves[0], halves[0], halves[1], halves[1]]

    bt = 2048
    T = B // bt

    numT = jnp.swapaxes(numerical_data, 0, 1)
    nst = pl.pallas_call(
        _num_stats_body,
        grid=(T,),
        in_specs=[pl.BlockSpec((NUM, bt), lambda i: (0, i))],
        out_specs=pl.BlockSpec((NUM, 2), lambda i: (0, 0)),
        out_shape=jax.ShapeDtypeStruct((NUM, 2), jnp.float32),
    )(numT)

    w1et = jnp.pad(W1[:, :ED].T, ((0, _FPAD * D - ED), (0, 0)))

    h1, st1 = pl.pallas_call(
        functools.partial(_l1_body, nB=fB),
        grid=(T,),
        in_specs=[
            pl.BlockSpec((bt, 128), lambda i: (i, 0)),
            pl.BlockSpec((bt, 128), lambda i: ((B // bt) + i, 0)),
            pl.BlockSpec((bt, 128), lambda i: (i, 0)),
            pl.BlockSpec((bt, 128), lambda i: ((B // bt) + i, 0)),
            pl.BlockSpec((NUM, bt), lambda i: (0, i)),
            pl.BlockSpec((NUM, 2), lambda i: (0, 0)),
            pl.BlockSpec((NUM, 1), lambda i: (0, 0)),
            pl.BlockSpec((NUM, 1), lambda i: (0, 0)),
            pl.BlockSpec((_FPAD * D, H1), lambda i: (0, 0)),
            pl.BlockSpec((NUM, H1), lambda i: (0, 0)),
            pl.BlockSpec((1, H1), lambda i: (0, 0)),
        ],
        out_specs=[
            pl.BlockSpec((bt, H1), lambda i: (i, 0)),
            pl.BlockSpec((2, H1), lambda i: (0, 0)),
        ],
        out_shape=[
            jax.ShapeDtypeStruct((B, H1), jnp.float32),
            jax.ShapeDtypeStruct((2, H1), jnp.float32),
        ],
    )(embs[0], embs[1], embs[2], embs[3], numT, nst, g0.reshape(NUM, 1),
      be0.reshape(NUM, 1), w1et.astype(jnp.bfloat16), W1[:, ED:].T,
      b1.reshape(1, H1))

    bt2 = 2048
    T2 = B // bt2
    h2, st2 = pl.pallas_call(
        functools.partial(_l2_body, nB=fB),
        grid=(T2,),
        in_specs=[
            pl.BlockSpec((bt2, H1), lambda i: (i, 0)),
            pl.BlockSpec((2, H1), lambda i: (0, 0)),
            pl.BlockSpec((1, H1), lambda i: (0, 0)),
            pl.BlockSpec((1, H1), lambda i: (0, 0)),
            pl.BlockSpec((H1, H2), lambda i: (0, 0)),
            pl.BlockSpec((1, H2), lambda i: (0, 0)),
        ],
        out_specs=[
            pl.BlockSpec((bt2, H2), lambda i: (i, 0)),
            pl.BlockSpec((2, H2), lambda i: (0, 0)),
        ],
        out_shape=[
            jax.ShapeDtypeStruct((B, H2), jnp.float32),
            jax.ShapeDtypeStruct((2, H2), jnp.float32),
        ],
    )(h1, st1, g1.reshape(1, H1), be1.reshape(1, H1), W2.T, b2.reshape(1, H2))

    outT = pl.pallas_call(
        functools.partial(_l3_body, nB=fB),
        grid=(T2,),
        in_specs=[
            pl.BlockSpec((bt2, H2), lambda i: (i, 0)),
            pl.BlockSpec((2, H2), lambda i: (0, 0)),
            pl.BlockSpec((1, H2), lambda i: (0, 0)),
            pl.BlockSpec((1, H2), lambda i: (0, 0)),
            pl.BlockSpec((1, H2), lambda i: (0, 0)),
            pl.BlockSpec((1, 1), lambda i: (0, 0)),
        ],
        out_specs=pl.BlockSpec((1, bt2), lambda i: (0, i)),
        out_shape=jax.ShapeDtypeStruct((1, B), jnp.float32),
    )(h2, st2, g2.reshape(1, H2), be2.reshape(1, H2), Wo.reshape(1, H2),
      bo.reshape(1, 1))

    return outT.reshape(B, 1)


# speedup vs baseline: 2.4318x; 2.4318x over previous
import functools

import jax
import jax.numpy as jnp
from jax import lax
from jax.experimental import pallas as pl
from jax.experimental.pallas import tpu as pltpu
from jax.experimental.pallas import tpu_sc as plsc

EPS = 1e-5
_NW = 32
_FPAD = 32
_NG = 4


_SC_MESH = plsc.VectorSubcoreMesh(core_axis_name="c", subcore_axis_name="s")
_SC_PARAMS = pltpu.CompilerParams(
    use_tc_tiling_on_sc=False, needs_layout_passes=False
)


def _sc_indices(catT, F, vpad):
    B = catT.shape[1]
    per_b = B // 8
    nidx = B * _FPAD

    @functools.partial(
        pl.kernel,
        mesh=_SC_MESH,
        compiler_params=_SC_PARAMS,
        out_type=jax.ShapeDtypeStruct((nidx,), jnp.int32),
        scratch_types=[
            pltpu.VMEM((8, per_b), jnp.int32),
            pltpu.VMEM((8 * per_b,), jnp.int32),
        ],
    )
    def idx_k(cat_hbm, out_hbm, cat_v, idx_v):
        wid = lax.axis_index("s") * 2 + lax.axis_index("c")
        k = wid // 8
        b0 = (wid % 8) * per_b
        frs = []
        for fj in range(8):
            fr = 8 * k + fj
            fr = jnp.where(fr < F, fr, (F - 2) + (fj & 1))
            frs.append(fr)
            pltpu.sync_copy(cat_hbm.at[fr, pl.ds(b0, per_b)], cat_v.at[fj])
        lanes = lax.iota(jnp.int32, 16)

        def body(cch, carry):
            for fj in range(8):
                v = cat_v[fj, pl.ds(cch * 16, 16)]
                u = v & 1023
                fl = frs[fj] - (k // 2) * 16
                t = (v - u) + ((v & 127) << 3) + (u >> 7) + fl * vpad
                pos = lanes * 8 + (cch * 128 + fj)
                plsc.store_scatter(idx_v, [pos], t)
            return carry

        lax.fori_loop(0, per_b // 16, body, 0)
        pltpu.sync_copy(idx_v, out_hbm.at[pl.ds(wid * 8 * per_b, 8 * per_b)])

    return idx_k(catT)


def _sc_gather(table, idx, C, nidx, gbase):
    Dd = table.shape[1]
    per_w = nidx // _NW
    n_chunk = per_w // C

    @functools.partial(
        pl.kernel,
        mesh=_SC_MESH,
        compiler_params=_SC_PARAMS,
        out_type=jax.ShapeDtypeStruct((nidx, Dd), jnp.float32),
        scratch_types=[
            pltpu.VMEM((per_w,), jnp.int32),
            pltpu.VMEM((2, C, Dd), jnp.float32),
            pltpu.SemaphoreType.DMA,
            pltpu.SemaphoreType.DMA,
        ],
    )
    def gather_k(table_hbm, idx_hbm, out_hbm, idx_v, rows_v, gsem0, gsem1):
        wid = lax.axis_index("s") * 2 + lax.axis_index("c")
        base = wid * per_w
        pltpu.sync_copy(idx_hbm.at[pl.ds(gbase + base, per_w)], idx_v)
        sems = (gsem0, gsem1)
        cur = pltpu.async_copy(
            table_hbm.at[idx_v.at[pl.ds(0, C)]], rows_v.at[0], sems[0]
        )
        for j in range(n_chunk):
            nxt = None
            if j + 1 < n_chunk:
                nxt = pltpu.async_copy(
                    table_hbm.at[idx_v.at[pl.ds((j + 1) * C, C)]],
                    rows_v.at[(j + 1) % 2],
                    sems[(j + 1) % 2],
                )
            cur.wait()
            pltpu.sync_copy(rows_v.at[j % 2], out_hbm.at[pl.ds(base + j * C, C)])
            cur = nxt

    return gather_k(table, idx)


_VC = 50176


def _tr_body(tt_ref, out_ref):
    x = tt_ref[0]
    for c in range(_VC // 1024):
        w = jnp.concatenate(
            [x[:, c * 1024 + q * 128 : c * 1024 + (q + 1) * 128] for q in range(8)],
            axis=0,
        )
        out_ref[pl.ds(c * 128, 128), :] = w.T


def _num_stats_body(numT_ref, acc_ref):
    x = numT_ref[...]
    stats = jnp.concatenate(
        [jnp.sum(x, axis=1, keepdims=True), jnp.sum(x * x, axis=1, keepdims=True)],
        axis=1,
    )
    i = pl.program_id(0)

    @pl.when(i == 0)
    def _():
        acc_ref[...] = stats

    @pl.when(i > 0)
    def _():
        acc_ref[...] += stats


def _l1_body(e0_ref, e1_ref, e2_ref, e3_ref, numT_ref, nst_ref, g0_ref,
             be0_ref, w1_ref, w1n_ref, b1_ref, h1_ref, acc_ref, *, nB):
    i = pl.program_id(0)
    m = nst_ref[:, 0:1] * (1.0 / nB)
    var = nst_ref[:, 1:2] * (1.0 / nB) - m * m
    numn = (numT_ref[...] - m) * lax.rsqrt(var + EPS) * g0_ref[...] + be0_ref[...]
    h = lax.dot_general(
        numn, w1n_ref[...], (((0,), (0,)), ((), ())),
        preferred_element_type=jnp.float32,
    ) + b1_ref[...]
    for k, e_ref in enumerate((e0_ref, e1_ref, e2_ref, e3_ref)):
        h += jnp.dot(
            e_ref[...], w1_ref[pl.ds(k * 128, 128), :],
            preferred_element_type=jnp.float32,
        )
    h = jnp.maximum(h, 0.0)
    h1_ref[...] = h
    stats = jnp.concatenate(
        [jnp.sum(h, axis=0, keepdims=True), jnp.sum(h * h, axis=0, keepdims=True)],
        axis=0,
    )

    @pl.when(i == 0)
    def _():
        acc_ref[...] = stats

    @pl.when(i > 0)
    def _():
        acc_ref[...] += stats


def _l2_body(h1_ref, st_ref, g_ref, be_ref, w2_ref, b2_ref, h2_ref, acc_ref, *, nB):
    i = pl.program_id(0)
    mean = st_ref[0:1, :] * (1.0 / nB)
    var = st_ref[1:2, :] * (1.0 / nB) - mean * mean
    xn = (h1_ref[...] - mean) * lax.rsqrt(var + EPS) * g_ref[...] + be_ref[...]
    h = jnp.dot(xn, w2_ref[...], preferred_element_type=jnp.float32)
    h = jnp.maximum(h + b2_ref[...], 0.0)
    h2_ref[...] = h
    stats = jnp.concatenate(
        [jnp.sum(h, axis=0, keepdims=True), jnp.sum(h * h, axis=0, keepdims=True)],
        axis=0,
    )

    @pl.when(i == 0)
    def _():
        acc_ref[...] = stats

    @pl.when(i > 0)
    def _():
        acc_ref[...] += stats


def _l3_body(h2_ref, st_ref, g_ref, be_ref, wo_ref, bo_ref, out_ref, *, nB):
    mean = st_ref[0:1, :] * (1.0 / nB)
    var = st_ref[1:2, :] * (1.0 / nB) - mean * mean
    xn = (h2_ref[...] - mean) * lax.rsqrt(var + EPS) * g_ref[...] + be_ref[...]
    out_ref[...] = lax.dot_general(
        wo_ref[...], xn, (((1,), (1,)), ((), ())),
        preferred_element_type=jnp.float32,
    ) + bo_ref[...]


def kernel(numerical_data, cat_data, tables, W1, b1, W2, b2, Wo, bo,
           g0, be0, g1, be1, g2, be2):
    B, NUM = numerical_data.shape
    F = cat_data.shape[1]
    V = tables.shape[1]
    D = tables.shape[2]
    ED = F * D
    GW = _FPAD // _NG
    H1, H2 = W1.shape[0], W2.shape[0]
    fB = float(B)

    tt = jnp.swapaxes(tables, 1, 2)
    nch = (V + _VC - 1) // _VC
    vpad = nch * _VC

    catT = jnp.swapaxes(cat_data, 0, 1)
    idx_r = _sc_indices(catT, F, vpad)

    halves = []
    for h in range(2):
        f0 = h * 2 * GW
        nf = min(2 * GW, F - f0)
        tp = pl.pallas_call(
            _tr_body,
            grid=(nf, nch),
            in_specs=[
                pl.BlockSpec((1, D, _VC), lambda f, c, f0=f0: (f0 + f, 0, c))
            ],
            out_specs=pl.BlockSpec((_VC // 8, 128), lambda f, c: (f * nch + c, 0)),
            out_shape=jax.ShapeDtypeStruct((nf * vpad // 8, 128), jnp.float32),
        )(tt)
        emb_h = _sc_gather(
            tp.reshape(nf * vpad, D), idx_r, C=2048,
            nidx=2 * B * GW, gbase=h * 2 * B * GW,
        )
        halves.append(emb_h.reshape(2 * B, GW * D))
    embs = [halves[0], halves[0], halves[1], halves[1]]

    bt = 2048
    T = B // bt

    numT = jnp.swapaxes(numerical_data, 0, 1)
    nst = pl.pallas_call(
        _num_stats_body,
        grid=(T,),
        in_specs=[pl.BlockSpec((NUM, bt), lambda i: (0, i))],
        out_specs=pl.BlockSpec((NUM, 2), lambda i: (0, 0)),
        out_shape=jax.ShapeDtypeStruct((NUM, 2), jnp.float32),
    )(numT)

    w1et = jnp.pad(W1[:, :ED].T, ((0, _FPAD * D - ED), (0, 0)))

    h1, st1 = pl.pallas_call(
        functools.partial(_l1_body, nB=fB),
        grid=(T,),
        in_specs=[
            pl.BlockSpec((bt, 128), lambda i: (i, 0)),
            pl.BlockSpec((bt, 128), lambda i: ((B // bt) + i, 0)),
            pl.BlockSpec((bt, 128), lambda i: (i, 0)),
            pl.BlockSpec((bt, 128), lambda i: ((B // bt) + i, 0)),
            pl.BlockSpec((NUM, bt), lambda i: (0, i)),
            pl.BlockSpec((NUM, 2), lambda i: (0, 0)),
            pl.BlockSpec((NUM, 1), lambda i: (0, 0)),
            pl.BlockSpec((NUM, 1), lambda i: (0, 0)),
            pl.BlockSpec((_FPAD * D, H1), lambda i: (0, 0)),
            pl.BlockSpec((NUM, H1), lambda i: (0, 0)),
            pl.BlockSpec((1, H1), lambda i: (0, 0)),
        ],
        out_specs=[
            pl.BlockSpec((bt, H1), lambda i: (i, 0)),
            pl.BlockSpec((2, H1), lambda i: (0, 0)),
        ],
        out_shape=[
            jax.ShapeDtypeStruct((B, H1), jnp.float32),
            jax.ShapeDtypeStruct((2, H1), jnp.float32),
        ],
    )(embs[0], embs[1], embs[2], embs[3], numT, nst, g0.reshape(NUM, 1),
      be0.reshape(NUM, 1), w1et, W1[:, ED:].T, b1.reshape(1, H1))

    bt2 = 2048
    T2 = B // bt2
    h2, st2 = pl.pallas_call(
        functools.partial(_l2_body, nB=fB),
        grid=(T2,),
        in_specs=[
            pl.BlockSpec((bt2, H1), lambda i: (i, 0)),
            pl.BlockSpec((2, H1), lambda i: (0, 0)),
            pl.BlockSpec((1, H1), lambda i: (0, 0)),
            pl.BlockSpec((1, H1), lambda i: (0, 0)),
            pl.BlockSpec((H1, H2), lambda i: (0, 0)),
            pl.BlockSpec((1, H2), lambda i: (0, 0)),
        ],
        out_specs=[
            pl.BlockSpec((bt2, H2), lambda i: (i, 0)),
            pl.BlockSpec((2, H2), lambda i: (0, 0)),
        ],
        out_shape=[
            jax.ShapeDtypeStruct((B, H2), jnp.float32),
            jax.ShapeDtypeStruct((2, H2), jnp.float32),
        ],
    )(h1, st1, g1.reshape(1, H1), be1.reshape(1, H1), W2.T, b2.reshape(1, H2))

    outT = pl.pallas_call(
        functools.partial(_l3_body, nB=fB),
        grid=(T2,),
        in_specs=[
            pl.BlockSpec((bt2, H2), lambda i: (i, 0)),
            pl.BlockSpec((2, H2), lambda i: (0, 0)),
            pl.BlockSpec((1, H2), lambda i: (0, 0)),
            pl.BlockSpec((1, H2), lambda i: (0, 0)),
            pl.BlockSpec((1, H2), lambda i: (0, 0)),
            pl.BlockSpec((1, 1), lambda i: (0, 0)),
        ],
        out_specs=pl.BlockSpec((1, bt2), lambda i: (0, i)),
        out_shape=jax.ShapeDtypeStruct((1, B), jnp.float32),
    )(h2, st2, g2.reshape(1, H2), be2.reshape(1, H2), Wo.reshape(1, H2),
      bo.reshape(1, 1))

    return outT.reshape(B, 1)
